# baseline (speedup 1.0000x reference)
import jax
import jax.numpy as jnp
from jax.experimental import pallas as pl
from jax.experimental.pallas import tpu as pltpu

BATCH = 16384
MAX_LEN = 150
GRID = 2
BLOCK_COLS = BATCH // GRID


def _body(x_ref, ids_ref, mask_ref, type_ref):
    x = x_ref[...]
    ids_ref[...] = x
    mask_ref[...] = jnp.where(x == 0, 0, 1).astype(jnp.int32)
    type_ref[...] = jnp.zeros_like(x)


def kernel(inputs):
    xt = inputs.T
    spec = pl.BlockSpec((MAX_LEN, BLOCK_COLS), lambda i: (0, i))
    out_shape = jax.ShapeDtypeStruct((MAX_LEN, BATCH), jnp.int32)
    ids, mask, type_ids = pl.pallas_call(
        _body,
        grid=(GRID,),
        in_specs=[spec],
        out_specs=[spec, spec, spec],
        out_shape=[out_shape, out_shape, out_shape],
        compiler_params=pltpu.CompilerParams(
            dimension_semantics=("arbitrary",),
        ),
    )(xt)
    return (ids.T, mask.T, type_ids.T)
